# baseline (speedup 1.0000x reference)
import functools

import jax
import jax.numpy as jnp
from jax import lax
from jax.experimental import pallas as pl
from jax.experimental.pallas import tpu as pltpu
from jax.experimental.pallas import tpu_sc as plsc

B = 32
N_EXAMPLES = 50000
N_CLASSES = 64
LANES = 16

ROWS_PER_CHUNK = 400
CHUNK_ELEMS = ROWS_PER_CHUNK * N_CLASSES
N_CHUNKS = N_EXAMPLES // ROWS_PER_CHUNK
GROUPS_PER_CHUNK = ROWS_PER_CHUNK // LANES
BATCH_ELEMS = N_EXAMPLES * N_CLASSES


def _body(x_hbm, out_hbm, buf, hist, out_v):
    wid = lax.axis_index("s") * 2 + lax.axis_index("c")
    iota = lax.broadcasted_iota(jnp.int32, (LANES,), 0)
    zeros = jnp.zeros((LANES,), jnp.float32)
    ones = jnp.ones((LANES,), jnp.float32)

    for j in range(N_CLASSES):
        hist[pl.ds(j * LANES, LANES)] = zeros

    batch_off = wid * BATCH_ELEMS

    def chunk_body(i, _):
        pltpu.sync_copy(x_hbm.at[pl.ds(batch_off + i * CHUNK_ELEMS,
                                       CHUNK_ELEMS)], buf)

        def group_body(g, _):
            row_base = g * (LANES * N_CLASSES) + iota * N_CLASSES
            m = jnp.full((LANES,), -jnp.inf, jnp.float32)
            a = jnp.zeros((LANES,), jnp.int32)
            for c in range(N_CLASSES):
                v = plsc.load_gather(buf, [row_base + c])
                gt = v > m
                a = jnp.where(gt, jnp.int32(c), a)
                m = jnp.where(gt, v, m)
            plsc.addupdate_scatter(hist, [a * LANES + iota], ones)
            return 0

        return lax.fori_loop(0, GROUPS_PER_CHUNK, group_body, 0)

    lax.fori_loop(0, N_CHUNKS, chunk_body, 0)

    scale = jnp.float32(1.0 / N_EXAMPLES)
    for j in range(N_CLASSES // LANES):
        acc = jnp.zeros((LANES,), jnp.float32)
        for k in range(LANES):
            acc = acc + plsc.load_gather(hist, [(j * LANES + iota) * LANES + k])
        out_v[pl.ds(j * LANES, LANES)] = acc * scale

    pltpu.sync_copy(out_v, out_hbm.at[wid])


def kernel(input):
    x_flat = input.reshape(-1)
    mesh = plsc.VectorSubcoreMesh(core_axis_name="c", subcore_axis_name="s")
    k = functools.partial(
        pl.kernel,
        out_type=jax.ShapeDtypeStruct((B, N_CLASSES), jnp.float32),
        mesh=mesh,
        scratch_types=[
            pltpu.VMEM((CHUNK_ELEMS,), jnp.float32),
            pltpu.VMEM((N_CLASSES * LANES,), jnp.float32),
            pltpu.VMEM((N_CLASSES,), jnp.float32),
        ],
        compiler_params=pltpu.CompilerParams(needs_layout_passes=False),
    )(_body)
    return k(x_flat)


# speedup vs baseline: 1.2611x; 1.2611x over previous
import functools

import jax
import jax.numpy as jnp
from jax import lax
from jax.experimental import pallas as pl
from jax.experimental.pallas import tpu as pltpu
from jax.experimental.pallas import tpu_sc as plsc

B = 32
N_EXAMPLES = 50000
N_CLASSES = 64
LANES = 16

ROWS_PER_CHUNK = 400
CHUNK_ELEMS = ROWS_PER_CHUNK * N_CLASSES
GROUPS_PER_CHUNK = ROWS_PER_CHUNK // LANES
N_CHUNKS = N_EXAMPLES // ROWS_PER_CHUNK


def _body(x_hbm, out_hbm, buf0, buf1, hist, out_v, sem0, sem1):
    wid = lax.axis_index("s") * 2 + lax.axis_index("c")
    iota = lax.broadcasted_iota(jnp.int32, (LANES,), 0)
    zeros = jnp.zeros((LANES,), jnp.float32)
    ones = jnp.ones((LANES,), jnp.float32)

    for j in range(N_CLASSES):
        hist[pl.ds(j * LANES, LANES)] = zeros

    def start(i, buf, sem):
        pltpu.async_copy(x_hbm.at[wid, i], buf, sem)

    def wait(buf, sem):
        pltpu.make_async_copy(x_hbm.at[wid, 0], buf, sem).wait()

    def process(buf):
        def group_body(g, _):
            row_base = g * (LANES * N_CLASSES) + iota * N_CLASSES
            m = jnp.full((LANES,), -jnp.inf, jnp.float32)
            a = jnp.zeros((LANES,), jnp.int32)
            for c in range(N_CLASSES):
                v = plsc.load_gather(buf, [row_base + c])
                gt = v > m
                a = jnp.where(gt, jnp.int32(c), a)
                m = jnp.where(gt, v, m)
            plsc.addupdate_scatter(hist, [a * LANES + iota], ones)
            return 0

        lax.fori_loop(0, GROUPS_PER_CHUNK, group_body, 0)

    start(0, buf0, sem0)

    def pair_body(k, _):
        start(2 * k + 1, buf1, sem1)
        wait(buf0, sem0)
        process(buf0)
        start(2 * k + 2, buf0, sem0)
        wait(buf1, sem1)
        process(buf1)
        return 0

    lax.fori_loop(0, (N_CHUNKS - 1) // 2, pair_body, 0)
    wait(buf0, sem0)
    process(buf0)

    scale = jnp.float32(1.0 / N_EXAMPLES)
    for j in range(N_CLASSES // LANES):
        acc = jnp.zeros((LANES,), jnp.float32)
        for k in range(LANES):
            acc = acc + plsc.load_gather(hist, [(j * LANES + iota) * LANES + k])
        out_v[pl.ds(j * LANES, LANES)] = acc * scale

    pltpu.sync_copy(out_v, out_hbm.at[wid])


def kernel(input):
    x3 = input.reshape(B, N_CHUNKS, CHUNK_ELEMS)
    mesh = plsc.VectorSubcoreMesh(core_axis_name="c", subcore_axis_name="s")
    k = functools.partial(
        pl.kernel,
        out_type=jax.ShapeDtypeStruct((B, N_CLASSES), jnp.float32),
        mesh=mesh,
        scratch_types=[
            pltpu.VMEM((CHUNK_ELEMS,), jnp.float32),
            pltpu.VMEM((CHUNK_ELEMS,), jnp.float32),
            pltpu.VMEM((N_CLASSES * LANES,), jnp.float32),
            pltpu.VMEM((N_CLASSES,), jnp.float32),
            pltpu.SemaphoreType.DMA,
            pltpu.SemaphoreType.DMA,
        ],
        compiler_params=pltpu.CompilerParams(needs_layout_passes=False),
    )(_body)
    return k(x3)


# speedup vs baseline: 1.3943x; 1.1056x over previous
import functools

import jax
import jax.numpy as jnp
from jax import lax
from jax.experimental import pallas as pl
from jax.experimental.pallas import tpu as pltpu
from jax.experimental.pallas import tpu_sc as plsc

B = 32
N_EXAMPLES = 50000
N_CLASSES = 64
LANES = 16

ROWS_PER_CHUNK = 400
CHUNK_ELEMS = ROWS_PER_CHUNK * N_CLASSES
N_CHUNKS = N_EXAMPLES // ROWS_PER_CHUNK
ROW_UNROLL = 4


def _body(x_hbm, out_hbm, buf0, buf1, hist, out_v, sem0, sem1):
    wid = lax.axis_index("s") * 2 + lax.axis_index("c")
    iota = lax.broadcasted_iota(jnp.int32, (LANES,), 0)
    zeros = jnp.zeros((LANES,), jnp.float32)
    ones = jnp.ones((LANES,), jnp.float32)
    lane15 = iota == (LANES - 1)
    neg_big = jnp.full((LANES,), -N_CLASSES, jnp.int32)
    negs = [-(iota + q * LANES) for q in range(4)]

    for j in range(N_CLASSES // LANES):
        hist[pl.ds(j * LANES, LANES)] = zeros

    def start(i, buf, sem):
        pltpu.async_copy(x_hbm.at[wid, i], buf, sem)

    def wait(buf, sem):
        pltpu.make_async_copy(x_hbm.at[wid, 0], buf, sem).wait()

    def one_row(buf, off):
        q = [buf[pl.ds(off + k * LANES, LANES)] for k in range(4)]
        m4 = jnp.maximum(jnp.maximum(q[0], q[1]), jnp.maximum(q[2], q[3]))
        mc = plsc.cummax(m4)
        mb = plsc.cummax(lax.rev(mc, (0,)))
        cand = jnp.where(q[0] == mb, negs[0], neg_big)
        for k in range(1, 4):
            cand = jnp.maximum(cand, jnp.where(q[k] == mb, negs[k], neg_big))
        cm = plsc.cummax(cand)
        plsc.addupdate_scatter(hist, [-cm], ones, mask=lane15)

    def process(buf):
        def row_body(r, _):
            base = r * (ROW_UNROLL * N_CLASSES)
            for j in range(ROW_UNROLL):
                one_row(buf, base + j * N_CLASSES)
            return 0

        lax.fori_loop(0, ROWS_PER_CHUNK // ROW_UNROLL, row_body, 0)

    start(0, buf0, sem0)

    def pair_body(k, _):
        start(2 * k + 1, buf1, sem1)
        wait(buf0, sem0)
        process(buf0)
        start(2 * k + 2, buf0, sem0)
        wait(buf1, sem1)
        process(buf1)
        return 0

    lax.fori_loop(0, (N_CHUNKS - 1) // 2, pair_body, 0)
    wait(buf0, sem0)
    process(buf0)

    scale = jnp.float32(1.0 / N_EXAMPLES)
    for j in range(N_CLASSES // LANES):
        out_v[pl.ds(j * LANES, LANES)] = hist[pl.ds(j * LANES, LANES)] * scale

    pltpu.sync_copy(out_v, out_hbm.at[wid])


def kernel(input):
    x3 = input.reshape(B, N_CHUNKS, CHUNK_ELEMS)
    mesh = plsc.VectorSubcoreMesh(core_axis_name="c", subcore_axis_name="s")
    k = functools.partial(
        pl.kernel,
        out_type=jax.ShapeDtypeStruct((B, N_CLASSES), jnp.float32),
        mesh=mesh,
        scratch_types=[
            pltpu.VMEM((CHUNK_ELEMS,), jnp.float32),
            pltpu.VMEM((CHUNK_ELEMS,), jnp.float32),
            pltpu.VMEM((N_CLASSES,), jnp.float32),
            pltpu.VMEM((N_CLASSES,), jnp.float32),
            pltpu.SemaphoreType.DMA,
            pltpu.SemaphoreType.DMA,
        ],
        compiler_params=pltpu.CompilerParams(needs_layout_passes=False),
    )(_body)
    return k(x3)
